# baseline (speedup 1.0000x reference)
import functools

import jax
import jax.numpy as jnp
from jax import lax
from jax.experimental import pallas as pl
from jax.experimental.pallas import tpu as pltpu
from jax.experimental.pallas import tpu_sc as plsc

NUM_EMBEDDINGS = 1000000
FEATURES = 64
BATCH = 16384
HIST = 50

NC = 2
NS = 16
NW = NC * NS
TOTAL = BATCH * HIST
CH = 128
N_CHUNKS = TOTAL // (NW * CH)

_mesh = plsc.VectorSubcoreMesh(core_axis_name="c", subcore_axis_name="s",
                               num_cores=NC, num_subcores=NS)


@functools.partial(
    pl.kernel,
    out_type=jax.ShapeDtypeStruct((NW, N_CHUNKS, CH, FEATURES), jnp.float32),
    mesh=_mesh,
    scratch_types=[
        pltpu.VMEM((N_CHUNKS, CH), jnp.int32),
        pltpu.VMEM((CH, FEATURES), jnp.float32),
        pltpu.SemaphoreType.DMA,
    ],
    compiler_params=pltpu.CompilerParams(use_tc_tiling_on_sc=False),
)
def _gather_kernel(table_hbm, idx_hbm, out_hbm, idx_v, rows_v, sem):
    wid = lax.axis_index("s") * NC + lax.axis_index("c")
    pltpu.sync_copy(idx_hbm.at[wid], idx_v)

    @pl.loop(0, N_CHUNKS)
    def _chunk(j):
        pltpu.async_copy(table_hbm.at[idx_v.at[j]], rows_v, sem).wait()
        pltpu.sync_copy(rows_v, out_hbm.at[wid].at[j])


def kernel(inputs, embedding):
    idx = inputs.astype(jnp.int32).reshape(NW, N_CHUNKS, CH)
    out = _gather_kernel(embedding, idx)
    return out.reshape(BATCH, HIST, FEATURES)


# speedup vs baseline: 1.1091x; 1.1091x over previous
import functools

import jax
import jax.numpy as jnp
from jax import lax
from jax.experimental import pallas as pl
from jax.experimental.pallas import tpu as pltpu
from jax.experimental.pallas import tpu_sc as plsc

NUM_EMBEDDINGS = 1000000
FEATURES = 64
BATCH = 16384
HIST = 50

NC = 2
NS = 16
NW = NC * NS
TOTAL = BATCH * HIST
CH = 128
N_CHUNKS = TOTAL // (NW * CH)

NBUF = 8
LOOK = 4

_mesh = plsc.VectorSubcoreMesh(core_axis_name="c", subcore_axis_name="s",
                               num_cores=NC, num_subcores=NS)


@functools.partial(
    pl.kernel,
    out_type=jax.ShapeDtypeStruct((NW, N_CHUNKS, CH, FEATURES), jnp.float32),
    mesh=_mesh,
    scratch_types=[
        pltpu.VMEM((N_CHUNKS, CH), jnp.int32),
        pltpu.VMEM((NBUF, CH, FEATURES), jnp.float32),
        pltpu.SemaphoreType.DMA((NBUF,)),
        pltpu.SemaphoreType.DMA((NBUF,)),
    ],
    compiler_params=pltpu.CompilerParams(use_tc_tiling_on_sc=False),
)
def _gather_kernel(table_hbm, idx_hbm, out_hbm, idx_v, rows_v, gsem, wsem):
    wid = lax.axis_index("s") * NC + lax.axis_index("c")
    out_w = out_hbm.at[wid]
    pltpu.sync_copy(idx_hbm.at[wid], idx_v)

    def start_gather(g, b):
        pltpu.async_copy(table_hbm.at[idx_v.at[g]], rows_v.at[b], gsem.at[b])

    def wait_gather(g, b):
        pltpu.make_async_copy(table_hbm.at[idx_v.at[g]], rows_v.at[b],
                              gsem.at[b]).wait()

    def start_write(c, b):
        pltpu.async_copy(rows_v.at[b], out_w.at[c], wsem.at[b])

    def wait_write(c, b):
        pltpu.make_async_copy(rows_v.at[b], out_w.at[c], wsem.at[b]).wait()

    for g in range(LOOK):
        start_gather(g, g % NBUF)

    @pl.loop(0, N_CHUNKS)
    def _step(j):
        @pl.when(j < N_CHUNKS - LOOK)
        def _():
            g = j + LOOK
            bg = g % NBUF

            @pl.when(g >= NBUF)
            def _():
                wait_write(g - NBUF, bg)

            start_gather(g, bg)

        bc = j % NBUF
        wait_gather(j, bc)
        start_write(j, bc)

    for c in range(N_CHUNKS - NBUF, N_CHUNKS):
        wait_write(c, c % NBUF)


def kernel(inputs, embedding):
    idx = inputs.astype(jnp.int32).reshape(NW, N_CHUNKS, CH)
    out = _gather_kernel(embedding, idx)
    return out.reshape(BATCH, HIST, FEATURES)
